# baseline (speedup 1.0000x reference)
import functools

import jax
import jax.numpy as jnp
from jax import lax
from jax.experimental import pallas as pl
from jax.experimental.pallas import tpu as pltpu
from jax.experimental.pallas import tpu_sc as plsc


def _tc_body(fm_ref, ln_ref, fk_ref, fq_ref, out_ref, *, hw, bq):
    b = pl.program_id(0)

    f = fm_ref[0]
    sim = lax.dot_general(
        f, ln_ref[0], (((0,), (0,)), ((), ())),
        preferred_element_type=jnp.float32)
    fk = fk_ref[0]
    sim = jnp.where(fk == 1, -1e9, sim)
    m = jnp.max(sim, axis=1, keepdims=True)
    kio = lax.broadcasted_iota(jnp.int32, (bq, hw), 1)
    idx = jnp.min(jnp.where(sim == m, kio, hw), axis=1, keepdims=True)
    fq = fq_ref[0]
    base = b * (hw + 1)
    out_ref[0] = jnp.where(fq == 1, idx + base, hw + base)


def _match_indices(fm, ln, fk, fq, bq):
    bz, c2, hw = fm.shape
    return pl.pallas_call(
        functools.partial(_tc_body, hw=hw, bq=bq),
        grid=(bz, hw // bq),
        in_specs=[
            pl.BlockSpec((1, c2, bq), lambda b, q: (b, 0, q)),
            pl.BlockSpec((1, c2, hw), lambda b, q: (b, 0, 0)),
            pl.BlockSpec((1, 1, hw), lambda b, q: (b, 0, 0)),
            pl.BlockSpec((1, bq, 1), lambda b, q: (b, q, 0)),
        ],
        out_specs=pl.BlockSpec((1, bq, 1), lambda b, q: (b, q, 0)),
        out_shape=jax.ShapeDtypeStruct((bz, hw, 1), jnp.int32),
    )(fm, ln, fk, fq)


def _sc_gather(idx3, table):
    nw, nchunks, _ = idx3.shape
    d = table.shape[1]
    bpw = nchunks * 128
    mesh = plsc.VectorSubcoreMesh(core_axis_name="c", subcore_axis_name="s")
    info = plsc.get_sparse_core_info()

    @functools.partial(
        pl.kernel, mesh=mesh,
        out_type=jax.ShapeDtypeStruct((nw * bpw, d), jnp.float32),
        scratch_types=[
            pltpu.VMEM((nchunks, 128), jnp.int32),
            pltpu.VMEM((bpw, d), jnp.float32),
            pltpu.SemaphoreType.DMA,
        ],
    )
    def k(idx_hbm, table_hbm, out_hbm, idx_v, rows_v, sem):
        wid = lax.axis_index("s") * info.num_cores + lax.axis_index("c")
        pltpu.sync_copy(idx_hbm.at[wid], idx_v)
        copies = [
            pltpu.async_copy(
                table_hbm.at[idx_v.at[j]],
                rows_v.at[pl.ds(j * 128, 128)], sem)
            for j in range(nchunks)
        ]
        for c in copies:
            c.wait()
        pltpu.sync_copy(rows_v, out_hbm.at[pl.ds(wid * bpw, bpw)])

    return k(idx3, table)


def kernel(input, mask):
    bz, c, h, w = input.shape
    c2 = c // 2
    hw = h * w
    fm = input[:, :c2].reshape(bz, c2, hw)
    lt = input[:, c2:].reshape(bz, c2, hw)
    fk = mask.reshape(bz, 1, hw)
    fq = mask.reshape(bz, hw, 1)

    bar = lax.optimization_barrier
    latter_r = bar(lt.transpose(0, 2, 1))
    sq = bar(latter_r * latter_r)
    ssum = bar(jnp.sum(sq, axis=-1, keepdims=True))
    norm = bar(jnp.sqrt(ssum) + 1e-8)
    ln32 = bar(latter_r / norm)
    ln = ln32.transpose(0, 2, 1).astype(jnp.bfloat16)
    fm_bf = fm.astype(jnp.bfloat16)

    idx = _match_indices(fm_bf, ln, fk, fq, bq=256)
    idx3 = idx.reshape(32, -1, 128)

    table = jnp.concatenate(
        [lt, jnp.zeros((bz, c2, 1), input.dtype)], axis=2)
    table = table.transpose(0, 2, 1)
    table = jnp.pad(table, ((0, 0), (0, 0), (0, 128 - c2)))
    table = table.reshape(bz * (hw + 1), 128)

    shift = _sc_gather(idx3, table)[:, :c2]
    shift_img = shift.reshape(bz, hw, c2).transpose(0, 2, 1).reshape(
        bz, c2, h, w)
    return jnp.concatenate([input, shift_img], axis=1)
